# baseline (speedup 1.0000x reference)
import jax
import jax.numpy as jnp
from jax import lax
from jax.experimental import pallas as pl
from jax.experimental.pallas import tpu as pltpu
from jax.experimental.pallas import tpu_sc as plsc

_NUM_CORES = 2
_NUM_SUBCORES = 16
_NUM_WORKERS = _NUM_CORES * _NUM_SUBCORES
_LANES = 16


def kernel(x, index):
    num_indices = index.shape[0]
    value_dim = x.shape[1]
    pack = 128 // value_dim
    num_packed = x.shape[0] // pack
    index = index.astype(jnp.int32)
    xp = x.reshape(num_packed, 128)
    b_per_w = num_indices // _NUM_WORKERS

    mesh = plsc.VectorSubcoreMesh(core_axis_name="c", subcore_axis_name="s")

    @pl.kernel(
        out_type=jax.ShapeDtypeStruct((value_dim, num_indices), x.dtype),
        mesh=mesh,
        compiler_params=pltpu.CompilerParams(needs_layout_passes=False),
        scratch_types=[
            pltpu.VMEM((b_per_w,), jnp.int32),
            pltpu.VMEM((b_per_w, 128), x.dtype),
            pltpu.VMEM((value_dim, b_per_w), x.dtype),
            pltpu.SemaphoreType.DMA,
        ],
    )
    def gather_kernel(xp_hbm, idx_hbm, ot_hbm, idx_v, rows_v, cols_v, sem):
        wid = lax.axis_index("s") * _NUM_CORES + lax.axis_index("c")
        base = wid * b_per_w
        pltpu.sync_copy(idx_hbm.at[pl.ds(base, b_per_w)], idx_v)

        @pl.loop(0, b_per_w, step=_LANES)
        def _(j0):
            qvec = jax.lax.shift_right_logical(idx_v[pl.ds(j0, _LANES)], 2)
            for k in range(_LANES):
                pltpu.async_copy(xp_hbm.at[pl.ds(qvec[k], 1)],
                                 rows_v.at[pl.ds(j0 + k, 1)], sem)

        pltpu.make_async_copy(xp_hbm.at[pl.ds(0, b_per_w)], rows_v, sem).wait()

        iota = lax.iota(jnp.int32, _LANES)

        @pl.loop(0, b_per_w, step=_LANES)
        def _(j0):
            ovec = (idx_v[pl.ds(j0, _LANES)] & 3) * value_dim
            for k in range(_LANES):
                j = j0 + k
                jvec = jnp.full((_LANES,), j, jnp.int32)
                off = jnp.full((_LANES,), ovec[k], jnp.int32)
                for h in range(value_dim // _LANES):
                    vals = plsc.load_gather(
                        rows_v, [jvec, off + (h * _LANES) + iota])
                    plsc.store_scatter(
                        cols_v, [(h * _LANES) + iota, jvec], vals)

        pltpu.sync_copy(cols_v, ot_hbm.at[:, pl.ds(base, b_per_w)])

    ot = gather_kernel(xp, index)
    return ot.T


# speedup vs baseline: 3.5470x; 3.5470x over previous
import jax
import jax.numpy as jnp
from jax import lax
from jax.experimental import pallas as pl
from jax.experimental.pallas import tpu as pltpu
from jax.experimental.pallas import tpu_sc as plsc

_NUM_CORES = 2
_NUM_SUBCORES = 16
_NUM_WORKERS = _NUM_CORES * _NUM_SUBCORES
_LANES = 16
_CHUNK = 16


def kernel(x, index):
    num_indices = index.shape[0]
    value_dim = x.shape[1]
    index = index.astype(jnp.int32)
    xt = x.T
    b_per_w = num_indices // _NUM_WORKERS

    mesh = plsc.VectorSubcoreMesh(core_axis_name="c", subcore_axis_name="s")

    @pl.kernel(
        out_type=jax.ShapeDtypeStruct((value_dim, num_indices), x.dtype),
        mesh=mesh,
        compiler_params=pltpu.CompilerParams(needs_layout_passes=False),
        scratch_types=[
            pltpu.VMEM((b_per_w,), jnp.int32),
            pltpu.VMEM((_CHUNK, value_dim, 128), x.dtype),
            pltpu.VMEM((value_dim, b_per_w), x.dtype),
            pltpu.SemaphoreType.DMA,
        ],
    )
    def gather_kernel(xt_hbm, idx_hbm, ot_hbm, idx_v, slab_v, cols_v, sem):
        wid = lax.axis_index("s") * _NUM_CORES + lax.axis_index("c")
        base = wid * b_per_w
        pltpu.sync_copy(idx_hbm.at[pl.ds(base, b_per_w)], idx_v)

        iota = lax.iota(jnp.int32, _LANES)

        @pl.loop(0, b_per_w, step=_CHUNK)
        def _(j0):
            ivec = idx_v[pl.ds(j0, _CHUNK)]
            qvec = (ivec >> 7) << 7
            rvec = ivec & 127
            for k in range(_CHUNK):
                start = pl.multiple_of(qvec[k], 128)
                pltpu.async_copy(xt_hbm.at[:, pl.ds(start, 128)],
                                 slab_v.at[k], sem)
            for k in range(_CHUNK):
                pltpu.make_async_copy(xt_hbm.at[:, pl.ds(0, 128)],
                                      slab_v.at[k], sem).wait()
            for k in range(_CHUNK):
                rbc = jnp.full((_LANES,), rvec[k], jnp.int32)
                jbc = jnp.full((_LANES,), j0 + k, jnp.int32)
                for h in range(value_dim // _LANES):
                    vals = plsc.load_gather(
                        slab_v.at[k], [(h * _LANES) + iota, rbc])
                    plsc.store_scatter(
                        cols_v, [(h * _LANES) + iota, jbc], vals)

        pltpu.sync_copy(cols_v, ot_hbm.at[:, pl.ds(base, b_per_w)])

    ot = gather_kernel(xt, index)
    return ot.T


# speedup vs baseline: 4.0535x; 1.1428x over previous
import jax
import jax.numpy as jnp
from jax import lax
from jax.experimental import pallas as pl
from jax.experimental.pallas import tpu as pltpu
from jax.experimental.pallas import tpu_sc as plsc

_NUM_CORES = 2
_NUM_SUBCORES = 16
_NUM_WORKERS = _NUM_CORES * _NUM_SUBCORES
_LANES = 16
_CHUNK = 16


def kernel(x, index):
    num_indices = index.shape[0]
    value_dim = x.shape[1]
    index = index.astype(jnp.int32)
    xt = x.T
    b_per_w = num_indices // _NUM_WORKERS
    n_chunks = b_per_w // _CHUNK

    mesh = plsc.VectorSubcoreMesh(core_axis_name="c", subcore_axis_name="s")

    @pl.kernel(
        out_type=jax.ShapeDtypeStruct((value_dim, num_indices), x.dtype),
        mesh=mesh,
        compiler_params=pltpu.CompilerParams(needs_layout_passes=False),
        scratch_types=[
            pltpu.VMEM((b_per_w,), jnp.int32),
            pltpu.VMEM((_CHUNK, value_dim, 128), x.dtype),
            pltpu.VMEM((value_dim, b_per_w), x.dtype),
        ] + [pltpu.SemaphoreType.DMA] * _CHUNK,
    )
    def gather_kernel(xt_hbm, idx_hbm, ot_hbm, idx_v, slab_v, cols_v, *sems):
        wid = lax.axis_index("s") * _NUM_CORES + lax.axis_index("c")
        base = wid * b_per_w
        pltpu.sync_copy(idx_hbm.at[pl.ds(base, b_per_w)], idx_v)

        iota = lax.iota(jnp.int32, _LANES)

        def fire(j0):
            qvec = (idx_v[pl.ds(j0, _CHUNK)] >> 7) << 7
            for k in range(_CHUNK):
                start = pl.multiple_of(qvec[k], 128)
                pltpu.async_copy(xt_hbm.at[:, pl.ds(start, 128)],
                                 slab_v.at[k], sems[k])

        def drain_extract(j0):
            rvec = idx_v[pl.ds(j0, _CHUNK)] & 127
            for k in range(_CHUNK):
                pltpu.make_async_copy(xt_hbm.at[:, pl.ds(0, 128)],
                                      slab_v.at[k], sems[k]).wait()
                rbc = jnp.full((_LANES,), rvec[k], jnp.int32)
                jbc = jnp.full((_LANES,), j0 + k, jnp.int32)
                for h in range(value_dim // _LANES):
                    vals = plsc.load_gather(
                        slab_v.at[k], [(h * _LANES) + iota, rbc])
                    plsc.store_scatter(
                        cols_v, [(h * _LANES) + iota, jbc], vals)

        fire(0)

        @pl.loop(1, n_chunks)
        def _(c):
            j0p = (c - 1) * _CHUNK
            j0n = c * _CHUNK
            rvec = idx_v[pl.ds(j0p, _CHUNK)] & 127
            qvec = (idx_v[pl.ds(j0n, _CHUNK)] >> 7) << 7
            for k in range(_CHUNK):
                pltpu.make_async_copy(xt_hbm.at[:, pl.ds(0, 128)],
                                      slab_v.at[k], sems[k]).wait()
                rbc = jnp.full((_LANES,), rvec[k], jnp.int32)
                jbc = jnp.full((_LANES,), j0p + k, jnp.int32)
                for h in range(value_dim // _LANES):
                    vals = plsc.load_gather(
                        slab_v.at[k], [(h * _LANES) + iota, rbc])
                    plsc.store_scatter(
                        cols_v, [(h * _LANES) + iota, jbc], vals)
                start = pl.multiple_of(qvec[k], 128)
                pltpu.async_copy(xt_hbm.at[:, pl.ds(start, 128)],
                                 slab_v.at[k], sems[k])

        drain_extract(b_per_w - _CHUNK)

        pltpu.sync_copy(cols_v, ot_hbm.at[:, pl.ds(base, b_per_w)])

    ot = gather_kernel(xt, index)
    return ot.T
